# baseline (speedup 1.0000x reference)
import functools
import math

import jax
import jax.numpy as jnp
from jax import lax
from jax.experimental import pallas as pl
from jax.experimental.pallas import tpu as pltpu
from jax.experimental.pallas import tpu_sc as plsc

_SEQ = 200
_GD = 300
_D = 768
_BI = 8

_NC = 2
_RPC = _SEQ // _NC


def _sc_gather_body(tokens_ref, table_ref, out_ref, tok_s, sem):
    cid = lax.axis_index("c")
    base = cid * _RPC

    pltpu.sync_copy(tokens_ref, tok_s)

    def start(r, c):
        pltpu.make_async_copy(
            table_ref.at[pl.ds(tok_s[base + r], 1), :],
            out_ref.at[pl.ds(base + r, 1), :],
            sem,
        ).start()
        return c

    lax.fori_loop(0, _RPC, start, 0)

    def wait(r, c):
        pltpu.make_async_copy(
            table_ref.at[pl.ds(0, 1), :],
            out_ref.at[pl.ds(base + r, 1), :],
            sem,
        ).wait()
        return c

    lax.fori_loop(0, _RPC, wait, 0)


def _tc_body(gath_ref, w_ref, b_ref, out_ref, vec):
    i = pl.program_id(0)

    @pl.when(i == 0)
    def _():
        vec[...] = (
            jnp.dot(gath_ref[...], w_ref[...], preferred_element_type=jnp.float32)
            + b_ref[...]
        )

    row = (i * _BI + jax.lax.broadcasted_iota(jnp.int32, (_BI, _D), 0)).astype(
        jnp.float32
    )
    col = jax.lax.broadcasted_iota(jnp.int32, (_BI, _D), 1)
    half = jnp.bitwise_and(col, -2).astype(jnp.float32)
    ang = row * jnp.exp(half * (-math.log(10000.0) / _D))
    pe = jnp.where(jnp.bitwise_and(col, 1) == 0, jnp.sin(ang), jnp.cos(ang))
    out_ref[...] = vec[...][None] + pe[:, None, :]


@jax.jit
def kernel(tokens, glove_table, W, b):
    gathered = pl.kernel(
        _sc_gather_body,
        mesh=plsc.ScalarSubcoreMesh(axis_name="c", num_cores=_NC),
        out_type=jax.ShapeDtypeStruct((_SEQ, _GD), jnp.float32),
        scratch_types=[
            pltpu.SMEM((_SEQ,), jnp.int32),
            pltpu.SemaphoreType.DMA,
        ],
    )(tokens, glove_table)

    b2 = b.reshape(1, _D)
    return pl.pallas_call(
        _tc_body,
        grid=(_SEQ // _BI,),
        in_specs=[
            pl.BlockSpec((_SEQ, _GD), lambda i: (0, 0)),
            pl.BlockSpec((_GD, _D), lambda i: (0, 0)),
            pl.BlockSpec((1, _D), lambda i: (0, 0)),
        ],
        out_specs=pl.BlockSpec((_BI, _SEQ, _D), lambda i: (i, 0, 0)),
        out_shape=jax.ShapeDtypeStruct((_SEQ, _SEQ, _D), jnp.float32),
        scratch_shapes=[
            pltpu.VMEM((_SEQ, _D), jnp.float32),
        ],
    )(gathered, W, b2)
